# Initial kernel scaffold; baseline (speedup 1.0000x reference)
import jax
import jax.numpy as jnp
from jax.experimental import pallas as pl


def kernel(x, edge_index, edge_weight, ids, W, att_src, att_dst, bias, ln_gamma, ln_beta, lin_W, lin_b):
    raise NotImplementedError("write your pallas kernel here")


# speedup vs baseline: 47.1035x; 47.1035x over previous
import functools

import jax
import jax.numpy as jnp
from jax import lax
from jax.experimental import pallas as pl
from jax.experimental.pallas import tpu as pltpu
from jax.experimental.pallas import tpu_sc as plsc

N_NODES = 50000
N_EDGES = 800000
IN_CH = 7
HEADS = 4
HID = 16
N_IDS = 8192
HC = HEADS * HID

NP = 50176
BLK = 1024
EPT = 50176
EK = 128
N_CHUNK = EPT // EK
IDS_PT = N_IDS // 16
PAD_NODE = NP - 1


def _one_hot_heads(dtype=jnp.float32):
    jj = lax.broadcasted_iota(jnp.int32, (HC, HEADS), 0)
    hh = lax.broadcasted_iota(jnp.int32, (HC, HEADS), 1)
    return ((jj // HID) == hh).astype(dtype)


def _prep_body(x_ref, w_ref, atts_ref, attd_ref,
               xwcat_ref, asrc_ref, adst_ref,
               asum0_ref, acc0cat_ref):
    S = _one_hot_heads()
    xw = jnp.dot(x_ref[...], w_ref[...], preferred_element_type=jnp.float32)
    asrc = jnp.dot(xw * atts_ref[...], S, preferred_element_type=jnp.float32)
    adst = jnp.dot(xw * attd_ref[...], S, preferred_element_type=jnp.float32)
    t = asrc + adst
    es = jnp.exp(jnp.maximum(t, 0.2 * t))
    acc0 = xw * jnp.dot(es, S.T, preferred_element_type=jnp.float32)
    xwcat_ref[0] = xw[:, :32]
    xwcat_ref[1] = xw[:, 32:]
    asrc_ref[...] = asrc
    adst_ref[...] = adst
    asum0_ref[...] = es
    acc0cat_ref[0] = acc0[:, :32]
    acc0cat_ref[1] = acc0[:, 32:]


def _run_prep(xp, W, att_src, att_dst):
    grid = NP // BLK
    rowspec = lambda w: pl.BlockSpec((BLK, w), lambda i: (i, 0))
    catspec = pl.BlockSpec((2, BLK, 32), lambda i: (0, i, 0))
    full = lambda a: pl.BlockSpec(a.shape, lambda i: (0,) * a.ndim)
    out_shapes = (
        jax.ShapeDtypeStruct((2, NP, 32), jnp.float32),
        jax.ShapeDtypeStruct((NP, HEADS), jnp.float32),
        jax.ShapeDtypeStruct((NP, HEADS), jnp.float32),
        jax.ShapeDtypeStruct((NP, HEADS), jnp.float32),
        jax.ShapeDtypeStruct((2, NP, 32), jnp.float32),
    )
    atts = att_src.reshape(1, HC)
    attd = att_dst.reshape(1, HC)
    return pl.pallas_call(
        _prep_body,
        grid=(grid,),
        in_specs=[rowspec(IN_CH), full(W), full(atts), full(attd)],
        out_specs=(catspec, rowspec(HEADS), rowspec(HEADS),
                   rowspec(HEADS), catspec),
        out_shape=out_shapes,
    )(xp, W, atts, attd)


def _edge_kernel(src_hbm, dst_hbm, asrc_hbm, adst_hbm,
                 xwcat_hbm, acc0cat_hbm, asum0_hbm, ids_hbm,
                 outacc_hbm, outasum_hbm,
                 accum_sp, asum_sp,
                 srcv, dstv, xsrcv, idsv, asr, adr, xwr, ebuf, msg,
                 orow, oasum, sem0, sem1):
    c = lax.axis_index("c")
    s = lax.axis_index("s")

    ini = s * (NP // 16)
    nrows = NP // 16
    pltpu.sync_copy(acc0cat_hbm.at[pl.ds(c * NP + ini, nrows)],
                    accum_sp.at[pl.ds(ini, nrows)])
    pltpu.sync_copy(asum0_hbm.at[pl.ds(ini, nrows)],
                    asum_sp.at[pl.ds(ini, nrows)])
    plsc.subcore_barrier()

    two_c = 2 * c
    io16 = lax.iota(jnp.int32, 16)
    row4 = io16 // 4
    col4 = io16 % 4

    def chunk_body(q, carry):
        ebase = s * EPT + q * EK
        pltpu.sync_copy(src_hbm.at[pl.ds(ebase, EK)], srcv)
        pltpu.sync_copy(dst_hbm.at[pl.ds(ebase, EK)], dstv)
        for t in range(EK // 16):
            xsrcv[pl.ds(16 * t, 16)] = srcv[pl.ds(16 * t, 16)] + c * NP
        pltpu.async_copy(asrc_hbm.at[srcv], asr, sem0).wait()
        pltpu.async_copy(adst_hbm.at[dstv], adr, sem0).wait()
        pltpu.async_copy(xwcat_hbm.at[xsrcv], xwr, sem1).wait()

        def quad_body(j, carry2):
            r4 = 4 * j + row4
            a_s = plsc.load_gather(asr, [r4, col4])
            a_d = plsc.load_gather(adr, [r4, col4])
            t = a_s + a_d
            e16 = jnp.exp(jnp.maximum(t, 0.2 * t))
            plsc.store_scatter(ebuf, [r4, col4], e16)
            for i in range(4):
                k = 4 * j + i
                kf = jnp.full((16,), k, jnp.int32)
                e0 = plsc.load_gather(ebuf, [kf, jnp.full((16,), two_c, jnp.int32)])
                e1 = plsc.load_gather(ebuf, [kf, jnp.full((16,), two_c + 1, jnp.int32)])
                x0 = plsc.load_gather(xwr, [kf, io16])
                x1 = plsc.load_gather(xwr, [kf, io16 + 16])
                plsc.store_scatter(msg, [kf, io16], x0 * e0)
                plsc.store_scatter(msg, [kf, io16 + 16], x1 * e1)
            return carry2
        lax.fori_loop(0, EK // 4, quad_body, 0, unroll=False)

        pltpu.sync_copy(msg, accum_sp.at[dstv], add=True)
        pltpu.sync_copy(ebuf, asum_sp.at[dstv], add=True)
        return carry
    lax.fori_loop(0, N_CHUNK, chunk_body, 0, unroll=False)

    plsc.subcore_barrier()

    def ids_body(u, carry):
        base = s * IDS_PT + u * EK
        pltpu.sync_copy(ids_hbm.at[pl.ds(base, EK)], idsv)
        pltpu.async_copy(accum_sp.at[idsv], orow, sem0).wait()
        pltpu.async_copy(asum_sp.at[idsv], oasum, sem1).wait()
        pltpu.sync_copy(orow, outacc_hbm.at[pl.ds(c * N_IDS + base, EK)])
        pltpu.sync_copy(oasum, outasum_hbm.at[pl.ds(c * N_IDS + base, EK)])
        return carry
    lax.fori_loop(0, IDS_PT // EK, ids_body, 0, unroll=False)


def _run_edges(srcp, dstp, asrc, adst, xwcat2, acc0cat2, asum0, ids):
    mesh = plsc.VectorSubcoreMesh(core_axis_name="c", subcore_axis_name="s")
    f32 = jnp.float32
    out_type = (
        jax.ShapeDtypeStruct((2 * N_IDS, 32), f32),
        jax.ShapeDtypeStruct((2 * N_IDS, HEADS), f32),
    )
    scratch = [
        pltpu.VMEM_SHARED((NP, 32), f32),
        pltpu.VMEM_SHARED((NP, HEADS), f32),
        pltpu.VMEM((EK,), jnp.int32),
        pltpu.VMEM((EK,), jnp.int32),
        pltpu.VMEM((EK,), jnp.int32),
        pltpu.VMEM((EK,), jnp.int32),
        pltpu.VMEM((EK, HEADS), f32),
        pltpu.VMEM((EK, HEADS), f32),
        pltpu.VMEM((EK, 32), f32),
        pltpu.VMEM((EK, HEADS), f32),
        pltpu.VMEM((EK, 32), f32),
        pltpu.VMEM((EK, 32), f32),
        pltpu.VMEM((EK, HEADS), f32),
        pltpu.SemaphoreType.DMA,
        pltpu.SemaphoreType.DMA,
    ]
    run = pl.kernel(_edge_kernel, out_type=out_type, mesh=mesh,
                    scratch_types=scratch,
                    compiler_params=pltpu.CompilerParams(
                        use_tc_tiling_on_sc=False,
                        needs_layout_passes=False))
    return run(srcp, dstp, asrc, adst, xwcat2, acc0cat2, asum0, ids)


def _finish_body(acc_ref, g_ref, bias_ref, gam_ref, bet_ref, lw_ref, lb_ref,
                 out_ref):
    S = _one_hot_heads()
    denom = jnp.dot(g_ref[...] + 1e-16, S.T, preferred_element_type=jnp.float32)
    out = acc_ref[...] / denom + bias_ref[...]
    mu = jnp.mean(out, axis=-1, keepdims=True)
    d = out - mu
    var = jnp.mean(d * d, axis=-1, keepdims=True)
    y = d / jnp.sqrt(var + 1e-5) * gam_ref[...] + bet_ref[...]
    logits = jnp.dot(y, lw_ref[...], preferred_element_type=jnp.float32) \
        + lb_ref[...]
    m = jnp.max(logits, axis=-1, keepdims=True)
    p = jnp.exp(logits - m)
    out_ref[...] = p / jnp.sum(p, axis=-1, keepdims=True)


def _run_finish(acc64, gsum, bias, ln_gamma, ln_beta, lin_W, lin_b):
    grid = 8
    rb = N_IDS // grid
    rowspec = lambda w: pl.BlockSpec((rb, w), lambda i: (i, 0))
    full = lambda a: pl.BlockSpec(a.shape, lambda i: (0,) * a.ndim)
    bias2 = bias.reshape(1, HC)
    gam2 = ln_gamma.reshape(1, HC)
    bet2 = ln_beta.reshape(1, HC)
    lb2 = lin_b.reshape(1, 7)
    return pl.pallas_call(
        _finish_body,
        grid=(grid,),
        in_specs=[rowspec(HC), rowspec(HEADS), full(bias2), full(gam2),
                  full(bet2), full(lin_W), full(lb2)],
        out_specs=rowspec(7),
        out_shape=jax.ShapeDtypeStruct((N_IDS, 7), jnp.float32),
    )(acc64, gsum, bias2, gam2, bet2, lin_W, lb2)


def kernel(x, edge_index, edge_weight, ids, W, att_src, att_dst, bias,
           ln_gamma, ln_beta, lin_W, lin_b):
    del edge_weight
    xp = jnp.pad(x, ((0, NP - N_NODES), (0, 0)))
    xwcat, asrc, adst, asum0, acc0cat = _run_prep(xp, W, att_src, att_dst)

    pad_e = 16 * EPT - N_EDGES
    padv = jnp.full((pad_e,), PAD_NODE, jnp.int32)
    srcp = jnp.concatenate([edge_index[0], padv])
    dstp = jnp.concatenate([edge_index[1], padv])

    outacc, outasum = _run_edges(
        srcp, dstp, asrc, adst, xwcat.reshape(2 * NP, 32),
        acc0cat.reshape(2 * NP, 32), asum0, ids)

    acc64 = jnp.concatenate([outacc[:N_IDS], outacc[N_IDS:]], axis=1)
    gsum = outasum[:N_IDS]
    return _run_finish(acc64, gsum, bias, ln_gamma, ln_beta, lin_W, lin_b)


# speedup vs baseline: 58.6188x; 1.2445x over previous
import functools

import jax
import jax.numpy as jnp
from jax import lax
from jax.experimental import pallas as pl
from jax.experimental.pallas import tpu as pltpu
from jax.experimental.pallas import tpu_sc as plsc

N_NODES = 50000
N_EDGES = 800000
IN_CH = 7
HEADS = 4
HID = 16
N_IDS = 8192
HC = HEADS * HID

NP = 50176
BLK = 1024
EPT = 50176
EK = 128
N_CHUNK = EPT // EK
IDS_PT = N_IDS // 16
PAD_NODE = NP - 1


def _one_hot_heads(dtype=jnp.float32):
    jj = lax.broadcasted_iota(jnp.int32, (HC, HEADS), 0)
    hh = lax.broadcasted_iota(jnp.int32, (HC, HEADS), 1)
    return ((jj // HID) == hh).astype(dtype)


def _prep_body(x_ref, w_ref, atts_ref, attd_ref,
               xwcat_ref, asrc_ref, adst_ref,
               asum0_ref, acc0cat_ref):
    S = _one_hot_heads()
    xw = jnp.dot(x_ref[...], w_ref[...], preferred_element_type=jnp.float32)
    asrc = jnp.dot(xw * atts_ref[...], S, preferred_element_type=jnp.float32)
    adst = jnp.dot(xw * attd_ref[...], S, preferred_element_type=jnp.float32)
    t = asrc + adst
    es = jnp.exp(jnp.maximum(t, 0.2 * t))
    acc0 = xw * jnp.dot(es, S.T, preferred_element_type=jnp.float32)
    xwcat_ref[0] = xw[:, :32]
    xwcat_ref[1] = xw[:, 32:]
    asrc_ref[...] = asrc
    adst_ref[...] = adst
    asum0_ref[0] = es[:, 0:2]
    asum0_ref[1] = es[:, 2:4]
    acc0cat_ref[0] = acc0[:, :32]
    acc0cat_ref[1] = acc0[:, 32:]


def _run_prep(xp, W, att_src, att_dst):
    grid = NP // BLK
    rowspec = lambda w: pl.BlockSpec((BLK, w), lambda i: (i, 0))
    catspec = pl.BlockSpec((2, BLK, 32), lambda i: (0, i, 0))
    full = lambda a: pl.BlockSpec(a.shape, lambda i: (0,) * a.ndim)
    out_shapes = (
        jax.ShapeDtypeStruct((2, NP, 32), jnp.float32),
        jax.ShapeDtypeStruct((NP, HEADS), jnp.float32),
        jax.ShapeDtypeStruct((NP, HEADS), jnp.float32),
        jax.ShapeDtypeStruct((2, NP, 2), jnp.float32),
        jax.ShapeDtypeStruct((2, NP, 32), jnp.float32),
    )
    atts = att_src.reshape(1, HC)
    attd = att_dst.reshape(1, HC)
    return pl.pallas_call(
        _prep_body,
        grid=(grid,),
        in_specs=[rowspec(IN_CH), full(W), full(atts), full(attd)],
        out_specs=(catspec, rowspec(HEADS), rowspec(HEADS),
                   pl.BlockSpec((2, BLK, 2), lambda i: (0, i, 0)), catspec),
        out_shape=out_shapes,
    )(xp, W, atts, attd)


def _edge_kernel(src_hbm, dst_hbm, asrc_hbm, adst_hbm,
                 xwcat_hbm, acc0cat_hbm, asum0_hbm, ids_hbm,
                 outacc_hbm, outasum_hbm,
                 accum_sp, asum_sp,
                 srcv, dstv, xsrcv, asr, adr, xwr, ebuf, msg, gsem, ssem,
                 idsv, orow, oasum, sem0):
    c = lax.axis_index("c")
    s = lax.axis_index("s")

    ini = s * (NP // 16)
    nrows = NP // 16
    pltpu.sync_copy(acc0cat_hbm.at[pl.ds(c * NP + ini, nrows)],
                    accum_sp.at[pl.ds(ini, nrows)])
    pltpu.sync_copy(asum0_hbm.at[pl.ds(c * NP + ini, nrows)],
                    asum_sp.at[pl.ds(ini, nrows)])
    plsc.subcore_barrier()

    two_c = 2 * c
    io16 = lax.iota(jnp.int32, 16)
    row4 = io16 // 4
    col4 = io16 % 4

    def chunk_body(q, carry):
        ebase = s * EPT + q * EK
        pltpu.sync_copy(src_hbm.at[pl.ds(ebase, EK)], srcv)
        pltpu.sync_copy(dst_hbm.at[pl.ds(ebase, EK)], dstv)
        for t in range(EK // 16):
            xsrcv[pl.ds(16 * t, 16)] = srcv[pl.ds(16 * t, 16)] + c * NP
        pltpu.async_copy(asrc_hbm.at[srcv], asr, gsem)
        pltpu.async_copy(adst_hbm.at[dstv], adr, gsem)
        pltpu.async_copy(xwcat_hbm.at[xsrcv], xwr, gsem)
        pltpu.make_async_copy(asrc_hbm.at[srcv], asr, gsem).wait()
        pltpu.make_async_copy(adst_hbm.at[dstv], adr, gsem).wait()
        pltpu.make_async_copy(xwcat_hbm.at[xsrcv], xwr, gsem).wait()

        def quad_body(j, carry2):
            r4 = 4 * j + row4
            a_s = plsc.load_gather(asr, [r4, col4])
            a_d = plsc.load_gather(adr, [r4, col4])
            t = a_s + a_d
            e16 = jnp.exp(jnp.maximum(t, 0.2 * t))
            col2 = jnp.clip(col4 - two_c, 0, 1)
            mym = (col4 - two_c == col2)
            plsc.store_scatter(ebuf, [r4, col2], e16, mask=mym)
            zero16 = jnp.zeros((16,), jnp.int32)
            for i in range(4):
                k = 4 * j + i
                kf = jnp.full((16,), k, jnp.int32)
                e0 = plsc.load_gather(ebuf, [kf, zero16])
                e1 = plsc.load_gather(ebuf, [kf, zero16 + 1])
                x0 = plsc.load_gather(xwr, [kf, io16])
                x1 = plsc.load_gather(xwr, [kf, io16 + 16])
                plsc.store_scatter(msg, [kf, io16], x0 * e0)
                plsc.store_scatter(msg, [kf, io16 + 16], x1 * e1)
            return carry2
        lax.fori_loop(0, EK // 4, quad_body, 0, unroll=False)

        pltpu.async_copy(msg, accum_sp.at[dstv], ssem, add=True)
        pltpu.async_copy(ebuf, asum_sp.at[dstv], ssem, add=True)
        pltpu.make_async_copy(msg, accum_sp.at[dstv], ssem).wait()
        pltpu.make_async_copy(ebuf, asum_sp.at[dstv], ssem).wait()
        return carry
    lax.fori_loop(0, N_CHUNK, chunk_body, 0, unroll=False)

    plsc.subcore_barrier()

    def ids_body(u, carry):
        base = s * IDS_PT + u * EK
        pltpu.sync_copy(ids_hbm.at[pl.ds(base, EK)], idsv)
        pltpu.async_copy(accum_sp.at[idsv], orow, sem0).wait()
        pltpu.async_copy(asum_sp.at[idsv], oasum, sem0).wait()
        pltpu.sync_copy(orow, outacc_hbm.at[pl.ds(c * N_IDS + base, EK)])
        pltpu.sync_copy(oasum, outasum_hbm.at[pl.ds(c * N_IDS + base, EK)])
        return carry
    lax.fori_loop(0, IDS_PT // EK, ids_body, 0, unroll=False)


def _run_edges(srcp, dstp, asrc, adst, xwcat2, acc0cat2, asum0, ids):
    mesh = plsc.VectorSubcoreMesh(core_axis_name="c", subcore_axis_name="s")
    f32 = jnp.float32
    out_type = (
        jax.ShapeDtypeStruct((2 * N_IDS, 32), f32),
        jax.ShapeDtypeStruct((2 * N_IDS, 2), f32),
    )
    scratch = [
        pltpu.VMEM_SHARED((NP, 32), f32),
        pltpu.VMEM_SHARED((NP, 2), f32),
        pltpu.VMEM((EK,), jnp.int32),
        pltpu.VMEM((EK,), jnp.int32),
        pltpu.VMEM((EK,), jnp.int32),
        pltpu.VMEM((EK, HEADS), f32),
        pltpu.VMEM((EK, HEADS), f32),
        pltpu.VMEM((EK, 32), f32),
        pltpu.VMEM((EK, 2), f32),
        pltpu.VMEM((EK, 32), f32),
        pltpu.SemaphoreType.DMA,
        pltpu.SemaphoreType.DMA,
        pltpu.VMEM((EK,), jnp.int32),
        pltpu.VMEM((EK, 32), f32),
        pltpu.VMEM((EK, 2), f32),
        pltpu.SemaphoreType.DMA,
    ]
    run = pl.kernel(_edge_kernel, out_type=out_type, mesh=mesh,
                    scratch_types=scratch,
                    compiler_params=pltpu.CompilerParams(
                        use_tc_tiling_on_sc=False,
                        needs_layout_passes=False))
    return run(srcp, dstp, asrc, adst, xwcat2, acc0cat2, asum0, ids)


def _finish_body(acc_ref, g_ref, bias_ref, gam_ref, bet_ref, lw_ref, lb_ref,
                 out_ref):
    S = _one_hot_heads()
    denom = jnp.dot(g_ref[...] + 1e-16, S.T, preferred_element_type=jnp.float32)
    out = acc_ref[...] / denom + bias_ref[...]
    mu = jnp.mean(out, axis=-1, keepdims=True)
    d = out - mu
    var = jnp.mean(d * d, axis=-1, keepdims=True)
    y = d / jnp.sqrt(var + 1e-5) * gam_ref[...] + bet_ref[...]
    logits = jnp.dot(y, lw_ref[...], preferred_element_type=jnp.float32) \
        + lb_ref[...]
    m = jnp.max(logits, axis=-1, keepdims=True)
    p = jnp.exp(logits - m)
    out_ref[...] = p / jnp.sum(p, axis=-1, keepdims=True)


def _run_finish(acc64, gsum, bias, ln_gamma, ln_beta, lin_W, lin_b):
    grid = 8
    rb = N_IDS // grid
    rowspec = lambda w: pl.BlockSpec((rb, w), lambda i: (i, 0))
    full = lambda a: pl.BlockSpec(a.shape, lambda i: (0,) * a.ndim)
    bias2 = bias.reshape(1, HC)
    gam2 = ln_gamma.reshape(1, HC)
    bet2 = ln_beta.reshape(1, HC)
    lb2 = lin_b.reshape(1, 7)
    return pl.pallas_call(
        _finish_body,
        grid=(grid,),
        in_specs=[rowspec(HC), rowspec(HEADS), full(bias2), full(gam2),
                  full(bet2), full(lin_W), full(lb2)],
        out_specs=rowspec(7),
        out_shape=jax.ShapeDtypeStruct((N_IDS, 7), jnp.float32),
    )(acc64, gsum, bias2, gam2, bet2, lin_W, lb2)


def kernel(x, edge_index, edge_weight, ids, W, att_src, att_dst, bias,
           ln_gamma, ln_beta, lin_W, lin_b):
    del edge_weight
    xp = jnp.pad(x, ((0, NP - N_NODES), (0, 0)))
    xwcat, asrc, adst, asum0, acc0cat = _run_prep(xp, W, att_src, att_dst)

    pad_e = 16 * EPT + EK - N_EDGES
    padv = jnp.full((pad_e,), PAD_NODE, jnp.int32)
    srcp = jnp.concatenate([edge_index[0], padv])
    dstp = jnp.concatenate([edge_index[1], padv])

    outacc, outasum = _run_edges(
        srcp, dstp, asrc, adst, xwcat.reshape(2 * NP, 32),
        acc0cat.reshape(2 * NP, 32), asum0.reshape(2 * NP, 2), ids)

    acc64 = jnp.concatenate([outacc[:N_IDS], outacc[N_IDS:]], axis=1)
    gsum = jnp.concatenate([outasum[:N_IDS], outasum[N_IDS:]], axis=1)
    return _run_finish(acc64, gsum, bias, ln_gamma, ln_beta, lin_W, lin_b)


# speedup vs baseline: 75.4945x; 1.2879x over previous
import functools

import jax
import jax.numpy as jnp
from jax import lax
from jax.experimental import pallas as pl
from jax.experimental.pallas import tpu as pltpu
from jax.experimental.pallas import tpu_sc as plsc

N_NODES = 50000
N_EDGES = 800000
IN_CH = 7
HEADS = 4
HID = 16
N_IDS = 8192
HC = HEADS * HID

NP = 50176
BLK = 1024
EPT = 50176
EK = 128
N_CHUNK = EPT // EK
IDS_PT = N_IDS // 16
PAD_NODE = NP - 1


def _one_hot_heads(dtype=jnp.float32):
    jj = lax.broadcasted_iota(jnp.int32, (HC, HEADS), 0)
    hh = lax.broadcasted_iota(jnp.int32, (HC, HEADS), 1)
    return ((jj // HID) == hh).astype(dtype)


def _prep_body(x_ref, w_ref, atts_ref, attd_ref,
               xwcat_ref, asrc_ref, adst_ref,
               asum0_ref, acc0cat_ref):
    S = _one_hot_heads()
    xw = jnp.dot(x_ref[...], w_ref[...], preferred_element_type=jnp.float32)
    asrc = jnp.dot(xw * atts_ref[...], S, preferred_element_type=jnp.float32)
    adst = jnp.dot(xw * attd_ref[...], S, preferred_element_type=jnp.float32)
    t = asrc + adst
    es = jnp.exp(jnp.maximum(t, 0.2 * t))
    acc0 = xw * jnp.dot(es, S.T, preferred_element_type=jnp.float32)
    xwcat_ref[0] = xw[:, :32]
    xwcat_ref[1] = xw[:, 32:]
    asrc_ref[...] = asrc
    adst_ref[...] = adst
    asum0_ref[0] = es[:, 0:2]
    asum0_ref[1] = es[:, 2:4]
    acc0cat_ref[0] = acc0[:, :32]
    acc0cat_ref[1] = acc0[:, 32:]


def _run_prep(xp, W, att_src, att_dst):
    grid = NP // BLK
    rowspec = lambda w: pl.BlockSpec((BLK, w), lambda i: (i, 0))
    catspec = pl.BlockSpec((2, BLK, 32), lambda i: (0, i, 0))
    full = lambda a: pl.BlockSpec(a.shape, lambda i: (0,) * a.ndim)
    out_shapes = (
        jax.ShapeDtypeStruct((2, NP, 32), jnp.float32),
        jax.ShapeDtypeStruct((NP, HEADS), jnp.float32),
        jax.ShapeDtypeStruct((NP, HEADS), jnp.float32),
        jax.ShapeDtypeStruct((2, NP, 2), jnp.float32),
        jax.ShapeDtypeStruct((2, NP, 32), jnp.float32),
    )
    atts = att_src.reshape(1, HC)
    attd = att_dst.reshape(1, HC)
    return pl.pallas_call(
        _prep_body,
        grid=(grid,),
        in_specs=[rowspec(IN_CH), full(W), full(atts), full(attd)],
        out_specs=(catspec, rowspec(HEADS), rowspec(HEADS),
                   pl.BlockSpec((2, BLK, 2), lambda i: (0, i, 0)), catspec),
        out_shape=out_shapes,
    )(xp, W, atts, attd)


def _edge_kernel(src_hbm, dst_hbm, asrc_hbm, adst_hbm,
                 xwcat_hbm, acc0cat_hbm, asum0_hbm, ids_hbm,
                 outacc_hbm, outasum_hbm,
                 accum_sp, asum_sp,
                 srcv, dstv, xsrcv, asr, adr, xwr, ebuf, msg, gsem, ssem,
                 idsv, orow, oasum, sem0):
    c = lax.axis_index("c")
    s = lax.axis_index("s")

    ini = s * (NP // 16)
    nrows = NP // 16
    pltpu.sync_copy(acc0cat_hbm.at[pl.ds(c * NP + ini, nrows)],
                    accum_sp.at[pl.ds(ini, nrows)])
    pltpu.sync_copy(asum0_hbm.at[pl.ds(c * NP + ini, nrows)],
                    asum_sp.at[pl.ds(ini, nrows)])
    plsc.subcore_barrier()

    two_c = 2 * c
    io16 = lax.iota(jnp.int32, 16)
    row4 = io16 // 4
    col4 = io16 % 4

    def stage_in(q, b):
        ebase = s * EPT + q * EK
        pltpu.sync_copy(src_hbm.at[pl.ds(ebase, EK)], srcv[b])
        pltpu.sync_copy(dst_hbm.at[pl.ds(ebase, EK)], dstv[b])
        for t in range(EK // 16):
            xsrcv[b][pl.ds(16 * t, 16)] = srcv[b][pl.ds(16 * t, 16)] + c * NP
        pltpu.async_copy(asrc_hbm.at[srcv[b]], asr[b], gsem[b])
        pltpu.async_copy(adst_hbm.at[dstv[b]], adr[b], gsem[b])
        pltpu.async_copy(xwcat_hbm.at[xsrcv[b]], xwr[b], gsem[b])

    def drain(b):
        pltpu.make_async_copy(asrc_hbm.at[srcv[b]], asr[b], gsem[b]).wait()
        pltpu.make_async_copy(adst_hbm.at[dstv[b]], adr[b], gsem[b]).wait()
        pltpu.make_async_copy(xwcat_hbm.at[xsrcv[b]], xwr[b], gsem[b]).wait()

    def compute(b):
        drain(b)

        def quad_body(j, carry2):
            r4 = 4 * j + row4
            a_s = plsc.load_gather(asr[b], [r4, col4])
            a_d = plsc.load_gather(adr[b], [r4, col4])
            t = a_s + a_d
            e16 = jnp.exp(jnp.maximum(t, 0.2 * t))
            col2 = jnp.clip(col4 - two_c, 0, 1)
            mym = (col4 - two_c == col2)
            plsc.store_scatter(ebuf, [r4, col2], e16, mask=mym)
            zero16 = jnp.zeros((16,), jnp.int32)
            for i in range(4):
                k = 4 * j + i
                kf = jnp.full((16,), k, jnp.int32)
                e0 = plsc.load_gather(ebuf, [kf, zero16])
                e1 = plsc.load_gather(ebuf, [kf, zero16 + 1])
                x0 = plsc.load_gather(xwr[b], [kf, io16])
                x1 = plsc.load_gather(xwr[b], [kf, io16 + 16])
                plsc.store_scatter(msg, [kf, io16], x0 * e0)
                plsc.store_scatter(msg, [kf, io16 + 16], x1 * e1)
            return carry2
        lax.fori_loop(0, EK // 4, quad_body, 0, unroll=False)

        pltpu.async_copy(msg, accum_sp.at[dstv[b]], ssem, add=True)
        pltpu.async_copy(ebuf, asum_sp.at[dstv[b]], ssem, add=True)
        pltpu.make_async_copy(msg, accum_sp.at[dstv[b]], ssem).wait()
        pltpu.make_async_copy(ebuf, asum_sp.at[dstv[b]], ssem).wait()

    stage_in(0, 0)

    def pair_body(i, carry):
        q0 = 2 * i
        stage_in(q0 + 1, 1)
        compute(0)
        stage_in(q0 + 2, 0)
        compute(1)
        return carry
    lax.fori_loop(0, N_CHUNK // 2, pair_body, 0, unroll=False)
    drain(0)

    plsc.subcore_barrier()

    def ids_body(u, carry):
        base = s * IDS_PT + u * EK
        pltpu.sync_copy(ids_hbm.at[pl.ds(base, EK)], idsv)
        pltpu.async_copy(accum_sp.at[idsv], orow, sem0).wait()
        pltpu.async_copy(asum_sp.at[idsv], oasum, sem0).wait()
        pltpu.sync_copy(orow, outacc_hbm.at[pl.ds(c * N_IDS + base, EK)])
        pltpu.sync_copy(oasum, outasum_hbm.at[pl.ds(c * N_IDS + base, EK)])
        return carry
    lax.fori_loop(0, IDS_PT // EK, ids_body, 0, unroll=False)


def _run_edges(srcp, dstp, asrc, adst, xwcat2, acc0cat2, asum0, ids):
    mesh = plsc.VectorSubcoreMesh(core_axis_name="c", subcore_axis_name="s")
    f32 = jnp.float32
    out_type = (
        jax.ShapeDtypeStruct((2 * N_IDS, 32), f32),
        jax.ShapeDtypeStruct((2 * N_IDS, 2), f32),
    )
    scratch = [
        pltpu.VMEM_SHARED((NP, 32), f32),
        pltpu.VMEM_SHARED((NP, 2), f32),
        [pltpu.VMEM((EK,), jnp.int32)] * 2,
        [pltpu.VMEM((EK,), jnp.int32)] * 2,
        [pltpu.VMEM((EK,), jnp.int32)] * 2,
        [pltpu.VMEM((EK, HEADS), f32)] * 2,
        [pltpu.VMEM((EK, HEADS), f32)] * 2,
        [pltpu.VMEM((EK, 32), f32)] * 2,
        pltpu.VMEM((EK, 2), f32),
        pltpu.VMEM((EK, 32), f32),
        [pltpu.SemaphoreType.DMA] * 2,
        pltpu.SemaphoreType.DMA,
        pltpu.VMEM((EK,), jnp.int32),
        pltpu.VMEM((EK, 32), f32),
        pltpu.VMEM((EK, 2), f32),
        pltpu.SemaphoreType.DMA,
    ]
    run = pl.kernel(_edge_kernel, out_type=out_type, mesh=mesh,
                    scratch_types=scratch,
                    compiler_params=pltpu.CompilerParams(
                        use_tc_tiling_on_sc=False,
                        needs_layout_passes=False))
    return run(srcp, dstp, asrc, adst, xwcat2, acc0cat2, asum0, ids)


def _finish_body(acc_ref, g_ref, bias_ref, gam_ref, bet_ref, lw_ref, lb_ref,
                 out_ref):
    S = _one_hot_heads()
    denom = jnp.dot(g_ref[...] + 1e-16, S.T, preferred_element_type=jnp.float32)
    out = acc_ref[...] / denom + bias_ref[...]
    mu = jnp.mean(out, axis=-1, keepdims=True)
    d = out - mu
    var = jnp.mean(d * d, axis=-1, keepdims=True)
    y = d / jnp.sqrt(var + 1e-5) * gam_ref[...] + bet_ref[...]
    logits = jnp.dot(y, lw_ref[...], preferred_element_type=jnp.float32) \
        + lb_ref[...]
    m = jnp.max(logits, axis=-1, keepdims=True)
    p = jnp.exp(logits - m)
    out_ref[...] = p / jnp.sum(p, axis=-1, keepdims=True)


def _run_finish(acc64, gsum, bias, ln_gamma, ln_beta, lin_W, lin_b):
    grid = 8
    rb = N_IDS // grid
    rowspec = lambda w: pl.BlockSpec((rb, w), lambda i: (i, 0))
    full = lambda a: pl.BlockSpec(a.shape, lambda i: (0,) * a.ndim)
    bias2 = bias.reshape(1, HC)
    gam2 = ln_gamma.reshape(1, HC)
    bet2 = ln_beta.reshape(1, HC)
    lb2 = lin_b.reshape(1, 7)
    return pl.pallas_call(
        _finish_body,
        grid=(grid,),
        in_specs=[rowspec(HC), rowspec(HEADS), full(bias2), full(gam2),
                  full(bet2), full(lin_W), full(lb2)],
        out_specs=rowspec(7),
        out_shape=jax.ShapeDtypeStruct((N_IDS, 7), jnp.float32),
    )(acc64, gsum, bias2, gam2, bet2, lin_W, lb2)


def kernel(x, edge_index, edge_weight, ids, W, att_src, att_dst, bias,
           ln_gamma, ln_beta, lin_W, lin_b):
    del edge_weight
    xp = jnp.pad(x, ((0, NP - N_NODES), (0, 0)))
    xwcat, asrc, adst, asum0, acc0cat = _run_prep(xp, W, att_src, att_dst)

    pad_e = 16 * EPT + EK - N_EDGES
    padv = jnp.full((pad_e,), PAD_NODE, jnp.int32)
    srcp = jnp.concatenate([edge_index[0], padv])
    dstp = jnp.concatenate([edge_index[1], padv])

    outacc, outasum = _run_edges(
        srcp, dstp, asrc, adst, xwcat.reshape(2 * NP, 32),
        acc0cat.reshape(2 * NP, 32), asum0.reshape(2 * NP, 2), ids)

    acc64 = jnp.concatenate([outacc[:N_IDS], outacc[N_IDS:]], axis=1)
    gsum = jnp.concatenate([outasum[:N_IDS], outasum[N_IDS:]], axis=1)
    return _run_finish(acc64, gsum, bias, ln_gamma, ln_beta, lin_W, lin_b)


# speedup vs baseline: 75.5896x; 1.0013x over previous
import functools

import jax
import jax.numpy as jnp
from jax import lax
from jax.experimental import pallas as pl
from jax.experimental.pallas import tpu as pltpu
from jax.experimental.pallas import tpu_sc as plsc

N_NODES = 50000
N_EDGES = 800000
IN_CH = 7
HEADS = 4
HID = 16
N_IDS = 8192
HC = HEADS * HID

NP = 50176
BLK = 1024
EPT = 50176
EK = 128
N_CHUNK = EPT // EK
IDS_PT = N_IDS // 16
PAD_NODE = NP - 1


def _one_hot_heads(dtype=jnp.float32):
    jj = lax.broadcasted_iota(jnp.int32, (HC, HEADS), 0)
    hh = lax.broadcasted_iota(jnp.int32, (HC, HEADS), 1)
    return ((jj // HID) == hh).astype(dtype)


def _prep_body(x_ref, w_ref, atts_ref, attd_ref,
               xwcat_ref, asrc_ref, adst_ref,
               asum0_ref, acc0cat_ref):
    S = _one_hot_heads()
    xw = jnp.dot(x_ref[...], w_ref[...], preferred_element_type=jnp.float32)
    asrc = jnp.dot(xw * atts_ref[...], S, preferred_element_type=jnp.float32)
    adst = jnp.dot(xw * attd_ref[...], S, preferred_element_type=jnp.float32)
    t = asrc + adst
    es = jnp.exp(jnp.maximum(t, 0.2 * t))
    acc0 = xw * jnp.dot(es, S.T, preferred_element_type=jnp.float32)
    xwcat_ref[0] = xw[:, :32]
    xwcat_ref[1] = xw[:, 32:]
    asrc_ref[...] = asrc
    adst_ref[...] = adst
    asum0_ref[0] = es[:, 0:2]
    asum0_ref[1] = es[:, 2:4]
    acc0cat_ref[0] = acc0[:, :32]
    acc0cat_ref[1] = acc0[:, 32:]


def _run_prep(xp, W, att_src, att_dst):
    grid = NP // BLK
    rowspec = lambda w: pl.BlockSpec((BLK, w), lambda i: (i, 0))
    catspec = pl.BlockSpec((2, BLK, 32), lambda i: (0, i, 0))
    full = lambda a: pl.BlockSpec(a.shape, lambda i: (0,) * a.ndim)
    out_shapes = (
        jax.ShapeDtypeStruct((2, NP, 32), jnp.float32),
        jax.ShapeDtypeStruct((NP, HEADS), jnp.float32),
        jax.ShapeDtypeStruct((NP, HEADS), jnp.float32),
        jax.ShapeDtypeStruct((2, NP, 2), jnp.float32),
        jax.ShapeDtypeStruct((2, NP, 32), jnp.float32),
    )
    atts = att_src.reshape(1, HC)
    attd = att_dst.reshape(1, HC)
    return pl.pallas_call(
        _prep_body,
        grid=(grid,),
        in_specs=[rowspec(IN_CH), full(W), full(atts), full(attd)],
        out_specs=(catspec, rowspec(HEADS), rowspec(HEADS),
                   pl.BlockSpec((2, BLK, 2), lambda i: (0, i, 0)), catspec),
        out_shape=out_shapes,
    )(xp, W, atts, attd)


def _edge_kernel(src_hbm, dst_hbm, asrc_hbm, adst_hbm,
                 xwcat_hbm, acc0cat_hbm, asum0_hbm, ids_hbm,
                 outacc_hbm, outasum_hbm,
                 accum_sp, asum_sp,
                 srcv, dstv, xsrcv, asr, adr, xwr, ebuf, msg, gsem, ssem,
                 idsv, orow, oasum, sem0):
    c = lax.axis_index("c")
    s = lax.axis_index("s")

    ini = s * (NP // 16)
    nrows = NP // 16
    pltpu.sync_copy(acc0cat_hbm.at[pl.ds(c * NP + ini, nrows)],
                    accum_sp.at[pl.ds(ini, nrows)])
    pltpu.sync_copy(asum0_hbm.at[pl.ds(c * NP + ini, nrows)],
                    asum_sp.at[pl.ds(ini, nrows)])
    plsc.subcore_barrier()

    two_c = 2 * c
    io16 = lax.iota(jnp.int32, 16)
    row4 = io16 // 4
    col4 = io16 % 4

    def stage_in(q, b):
        ebase = s * EPT + q * EK
        pltpu.sync_copy(src_hbm.at[pl.ds(ebase, EK)], srcv[b])
        pltpu.sync_copy(dst_hbm.at[pl.ds(ebase, EK)], dstv[b])
        for t in range(EK // 16):
            xsrcv[b][pl.ds(16 * t, 16)] = srcv[b][pl.ds(16 * t, 16)] + c * NP
        pltpu.async_copy(asrc_hbm.at[srcv[b]], asr[b], gsem[b])
        pltpu.async_copy(adst_hbm.at[dstv[b]], adr[b], gsem[b])
        pltpu.async_copy(xwcat_hbm.at[xsrcv[b]], xwr[b], gsem[b])

    def drain(b):
        pltpu.make_async_copy(asrc_hbm.at[srcv[b]], asr[b], gsem[b]).wait()
        pltpu.make_async_copy(adst_hbm.at[dstv[b]], adr[b], gsem[b]).wait()
        pltpu.make_async_copy(xwcat_hbm.at[xsrcv[b]], xwr[b], gsem[b]).wait()

    def compute(b):
        drain(b)

        def quad_body(j, carry2):
            r4 = 4 * j + row4
            a_s = plsc.load_gather(asr[b], [r4, col4])
            a_d = plsc.load_gather(adr[b], [r4, col4])
            t = a_s + a_d
            e16 = jnp.exp(jnp.maximum(t, 0.2 * t))
            col2 = jnp.clip(col4 - two_c, 0, 1)
            mym = (col4 - two_c == col2)
            plsc.store_scatter(ebuf, [r4, col2], e16, mask=mym)
            zero16 = jnp.zeros((16,), jnp.int32)
            for i in range(4):
                k = 4 * j + i
                kf = jnp.full((16,), k, jnp.int32)
                e0 = plsc.load_gather(ebuf, [kf, zero16])
                e1 = plsc.load_gather(ebuf, [kf, zero16 + 1])
                x0 = plsc.load_gather(xwr[b], [kf, io16])
                x1 = plsc.load_gather(xwr[b], [kf, io16 + 16])
                plsc.store_scatter(msg, [kf, io16], x0 * e0)
                plsc.store_scatter(msg, [kf, io16 + 16], x1 * e1)
            return carry2
        lax.fori_loop(0, EK // 4, quad_body, 0, unroll=2)

        pltpu.async_copy(msg, accum_sp.at[dstv[b]], ssem, add=True)
        pltpu.async_copy(ebuf, asum_sp.at[dstv[b]], ssem, add=True)
        pltpu.make_async_copy(msg, accum_sp.at[dstv[b]], ssem).wait()
        pltpu.make_async_copy(ebuf, asum_sp.at[dstv[b]], ssem).wait()

    stage_in(0, 0)

    def pair_body(i, carry):
        q0 = 2 * i
        stage_in(q0 + 1, 1)
        compute(0)
        stage_in(q0 + 2, 0)
        compute(1)
        return carry
    lax.fori_loop(0, N_CHUNK // 2, pair_body, 0, unroll=False)
    drain(0)

    plsc.subcore_barrier()

    def ids_body(u, carry):
        base = s * IDS_PT + u * EK
        pltpu.sync_copy(ids_hbm.at[pl.ds(base, EK)], idsv)
        pltpu.async_copy(accum_sp.at[idsv], orow, sem0).wait()
        pltpu.async_copy(asum_sp.at[idsv], oasum, sem0).wait()
        pltpu.sync_copy(orow, outacc_hbm.at[pl.ds(c * N_IDS + base, EK)])
        pltpu.sync_copy(oasum, outasum_hbm.at[pl.ds(c * N_IDS + base, EK)])
        return carry
    lax.fori_loop(0, IDS_PT // EK, ids_body, 0, unroll=False)


def _run_edges(srcp, dstp, asrc, adst, xwcat2, acc0cat2, asum0, ids):
    mesh = plsc.VectorSubcoreMesh(core_axis_name="c", subcore_axis_name="s")
    f32 = jnp.float32
    out_type = (
        jax.ShapeDtypeStruct((2 * N_IDS, 32), f32),
        jax.ShapeDtypeStruct((2 * N_IDS, 2), f32),
    )
    scratch = [
        pltpu.VMEM_SHARED((NP, 32), f32),
        pltpu.VMEM_SHARED((NP, 2), f32),
        [pltpu.VMEM((EK,), jnp.int32)] * 2,
        [pltpu.VMEM((EK,), jnp.int32)] * 2,
        [pltpu.VMEM((EK,), jnp.int32)] * 2,
        [pltpu.VMEM((EK, HEADS), f32)] * 2,
        [pltpu.VMEM((EK, HEADS), f32)] * 2,
        [pltpu.VMEM((EK, 32), f32)] * 2,
        pltpu.VMEM((EK, 2), f32),
        pltpu.VMEM((EK, 32), f32),
        [pltpu.SemaphoreType.DMA] * 2,
        pltpu.SemaphoreType.DMA,
        pltpu.VMEM((EK,), jnp.int32),
        pltpu.VMEM((EK, 32), f32),
        pltpu.VMEM((EK, 2), f32),
        pltpu.SemaphoreType.DMA,
    ]
    run = pl.kernel(_edge_kernel, out_type=out_type, mesh=mesh,
                    scratch_types=scratch,
                    compiler_params=pltpu.CompilerParams(
                        use_tc_tiling_on_sc=False,
                        needs_layout_passes=False))
    return run(srcp, dstp, asrc, adst, xwcat2, acc0cat2, asum0, ids)


def _finish_body(acc_ref, g_ref, bias_ref, gam_ref, bet_ref, lw_ref, lb_ref,
                 out_ref):
    S = _one_hot_heads()
    denom = jnp.dot(g_ref[...] + 1e-16, S.T, preferred_element_type=jnp.float32)
    out = acc_ref[...] / denom + bias_ref[...]
    mu = jnp.mean(out, axis=-1, keepdims=True)
    d = out - mu
    var = jnp.mean(d * d, axis=-1, keepdims=True)
    y = d / jnp.sqrt(var + 1e-5) * gam_ref[...] + bet_ref[...]
    logits = jnp.dot(y, lw_ref[...], preferred_element_type=jnp.float32) \
        + lb_ref[...]
    m = jnp.max(logits, axis=-1, keepdims=True)
    p = jnp.exp(logits - m)
    out_ref[...] = p / jnp.sum(p, axis=-1, keepdims=True)


def _run_finish(acc64, gsum, bias, ln_gamma, ln_beta, lin_W, lin_b):
    grid = 8
    rb = N_IDS // grid
    rowspec = lambda w: pl.BlockSpec((rb, w), lambda i: (i, 0))
    full = lambda a: pl.BlockSpec(a.shape, lambda i: (0,) * a.ndim)
    bias2 = bias.reshape(1, HC)
    gam2 = ln_gamma.reshape(1, HC)
    bet2 = ln_beta.reshape(1, HC)
    lb2 = lin_b.reshape(1, 7)
    return pl.pallas_call(
        _finish_body,
        grid=(grid,),
        in_specs=[rowspec(HC), rowspec(HEADS), full(bias2), full(gam2),
                  full(bet2), full(lin_W), full(lb2)],
        out_specs=rowspec(7),
        out_shape=jax.ShapeDtypeStruct((N_IDS, 7), jnp.float32),
    )(acc64, gsum, bias2, gam2, bet2, lin_W, lb2)


def kernel(x, edge_index, edge_weight, ids, W, att_src, att_dst, bias,
           ln_gamma, ln_beta, lin_W, lin_b):
    del edge_weight
    xp = jnp.pad(x, ((0, NP - N_NODES), (0, 0)))
    xwcat, asrc, adst, asum0, acc0cat = _run_prep(xp, W, att_src, att_dst)

    pad_e = 16 * EPT + EK - N_EDGES
    padv = jnp.full((pad_e,), PAD_NODE, jnp.int32)
    srcp = jnp.concatenate([edge_index[0], padv])
    dstp = jnp.concatenate([edge_index[1], padv])

    outacc, outasum = _run_edges(
        srcp, dstp, asrc, adst, xwcat.reshape(2 * NP, 32),
        acc0cat.reshape(2 * NP, 32), asum0.reshape(2 * NP, 2), ids)

    acc64 = jnp.concatenate([outacc[:N_IDS], outacc[N_IDS:]], axis=1)
    gsum = jnp.concatenate([outasum[:N_IDS], outasum[N_IDS:]], axis=1)
    return _run_finish(acc64, gsum, bias, ln_gamma, ln_beta, lin_W, lin_b)
